# baseline (speedup 1.0000x reference)
import functools

import jax
import jax.numpy as jnp
from jax import lax
from jax.experimental import pallas as pl
from jax.experimental.pallas import tpu as pltpu
from jax.experimental.pallas import tpu_sc as plsc


def _transpose_table(wt):
    D, V = wt.shape
    BLK = 32768
    HB = BLK // 2
    NB = (V + BLK - 1) // BLK

    def body(wt_ref, out_ref):
        x = wt_ref[...]
        out_ref[:, 0:D] = x[:, 0:HB].T
        out_ref[:, D:2 * D] = x[:, HB:BLK].T

    return pl.pallas_call(
        body,
        grid=(NB,),
        in_specs=[pl.BlockSpec((D, BLK), lambda i: (0, i))],
        out_specs=pl.BlockSpec((HB, 2 * D), lambda i: (i, 0)),
        out_shape=jax.ShapeDtypeStruct((NB * HB, 2 * D), jnp.float32),
    )(wt)


def _transpose_out(rows, half, prev, T1, B1, B0, D):
    def body(*refs):
        in_ref, out_ref = refs[0], refs[-1]
        out_ref[0] = in_ref[:, 0:D].T

    in_specs = [pl.BlockSpec((B0, 2 * D), lambda i: (i, 0))]
    args = (rows,)
    aliases = {}
    if prev is not None:
        in_specs.append(pl.BlockSpec(memory_space=pl.ANY))
        args = (rows, prev)
        aliases = {1: 0}
    return pl.pallas_call(
        body,
        grid=(T1,),
        in_specs=in_specs,
        out_specs=pl.BlockSpec((1, D, B0), lambda i: (i + half * T1, 0, 0)),
        out_shape=jax.ShapeDtypeStruct((B1, D, B0), jnp.float32),
        input_output_aliases=aliases,
    )(*args)


def _sc_gather(idx2, table, B, D):
    info = plsc.get_sparse_core_info()
    NC, NS = info.num_cores, info.num_subcores
    NW = NC * NS

    KW = 128
    CH = 5
    ROWS = CH * KW
    per_w = B // NW
    n_chunks = per_w // ROWS
    assert per_w % ROWS == 0 and n_chunks % 2 == 0
    H = n_chunks // 2
    idx_rows_w = per_w // KW

    mesh = plsc.VectorSubcoreMesh(core_axis_name="c", subcore_axis_name="s")

    @functools.partial(
        pl.kernel,
        mesh=mesh,
        out_type=jax.ShapeDtypeStruct((B, 2 * D), jnp.float32),
        scratch_types=[
            pltpu.VMEM((idx_rows_w, KW), jnp.int32),
            pltpu.VMEM((ROWS, D), jnp.float32),
            pltpu.VMEM((ROWS, D), jnp.float32),
            pltpu.SemaphoreType.DMA,
            pltpu.SemaphoreType.DMA,
            pltpu.SemaphoreType.DMA,
            pltpu.SemaphoreType.DMA,
        ],
        compiler_params=pltpu.CompilerParams(use_tc_tiling_on_sc=False),
    )
    def emb(idx_hbm, table_hbm, out_hbm, idx_v, rows0, rows1,
            g0, g1, o0, o1):
        wid = lax.axis_index("s") * NC + lax.axis_index("c")
        out_row0 = wid * per_w

        pltpu.sync_copy(idx_hbm.at[pl.ds(wid * idx_rows_w, idx_rows_w)],
                        idx_v)

        def fire_gathers(chunk, rows_v, sem):
            for i in range(CH):
                pltpu.make_async_copy(
                    table_hbm.at[idx_v.at[chunk * CH + i]],
                    rows_v.at[pl.ds(i * KW, KW), :],
                    sem,
                ).start()

        def drain_gathers(rows_v, sem):
            pltpu.make_async_copy(
                table_hbm.at[pl.ds(0, ROWS), :], rows_v, sem).wait()

        def fire_write(chunk, rows_v, sem):
            pltpu.make_async_copy(
                rows_v,
                out_hbm.at[pl.ds(out_row0 + chunk * ROWS, ROWS),
                           pl.ds(0, D)],
                sem,
            ).start()

        def drain_write(rows_v, sem):
            pltpu.make_async_copy(
                rows_v,
                out_hbm.at[pl.ds(0, ROWS), pl.ds(0, D)], sem).wait()

        fire_gathers(0, rows0, g0)
        fire_gathers(1, rows1, g1)

        def body(j, carry):
            drain_gathers(rows0, g0)
            fire_write(2 * j - 2, rows0, o0)
            drain_write(rows0, o0)
            fire_gathers(2 * j, rows0, g0)
            drain_gathers(rows1, g1)
            fire_write(2 * j - 1, rows1, o1)
            drain_write(rows1, o1)
            fire_gathers(2 * j + 1, rows1, g1)
            return carry

        lax.fori_loop(1, H, body, 0)

        drain_gathers(rows0, g0)
        fire_write(2 * H - 2, rows0, o0)
        drain_gathers(rows1, g1)
        fire_write(2 * H - 1, rows1, o1)
        drain_write(rows0, o0)
        drain_write(rows1, o1)

    return emb(idx2, table)


def kernel(input_tensor, weight):
    B0, B1 = input_tensor.shape
    V, D = weight.shape
    B = B0 * B1
    KW = 128
    NSPLIT = 20
    T1 = B1 // NSPLIT

    BLK = 32768
    HB = BLK // 2
    v = input_tensor.T.astype(jnp.int32)
    blk, off = v // BLK, v % BLK
    idx2 = (2 * (blk * HB + off % HB) + off // HB).reshape(B // KW, KW)
    split_rows = B // (NSPLIT * KW)

    w_t = weight.T
    table = _transpose_table(w_t)
    table2 = table.reshape(2 * table.shape[0], D)

    gs = [_sc_gather(idx2[h * split_rows:(h + 1) * split_rows],
                     table2, B // NSPLIT, D) for h in range(NSPLIT)]
    o = None
    for h in range(NSPLIT):
        o = _transpose_out(gs[h], h, o, T1, B1, B0, D)

    return jnp.transpose(o, (2, 0, 1))


# speedup vs baseline: 1.0136x; 1.0136x over previous
import functools

import jax
import jax.numpy as jnp
from jax import lax
from jax.experimental import pallas as pl
from jax.experimental.pallas import tpu as pltpu
from jax.experimental.pallas import tpu_sc as plsc


def _transpose_table(wt):
    D, V = wt.shape
    BLK = 32768
    HB = BLK // 2
    NB = (V + BLK - 1) // BLK

    def body(wt_ref, out_ref):
        x = wt_ref[...]
        out_ref[:, 0:D] = x[:, 0:HB].T
        out_ref[:, D:2 * D] = x[:, HB:BLK].T

    return pl.pallas_call(
        body,
        grid=(NB,),
        in_specs=[pl.BlockSpec((D, BLK), lambda i: (0, i))],
        out_specs=pl.BlockSpec((HB, 2 * D), lambda i: (i, 0)),
        out_shape=jax.ShapeDtypeStruct((NB * HB, 2 * D), jnp.float32),
    )(wt)


def _transpose_out(rows, half, prev, T1, B1, B0, D):
    def body(*refs):
        in_ref, out_ref = refs[0], refs[-1]
        out_ref[0] = in_ref[:, 0:D].T

    in_specs = [pl.BlockSpec((B0, 2 * D), lambda i: (i, 0))]
    args = (rows,)
    aliases = {}
    if prev is not None:
        in_specs.append(pl.BlockSpec(memory_space=pl.ANY))
        args = (rows, prev)
        aliases = {1: 0}
    return pl.pallas_call(
        body,
        grid=(T1,),
        in_specs=in_specs,
        out_specs=pl.BlockSpec((1, D, B0), lambda i: (i + half * T1, 0, 0)),
        out_shape=jax.ShapeDtypeStruct((B1, D, B0), jnp.float32),
        input_output_aliases=aliases,
    )(*args)


def _sc_gather(idx2, table, B, D):
    info = plsc.get_sparse_core_info()
    NC, NS = info.num_cores, info.num_subcores
    NW = NC * NS

    KW = 128
    CH = 5
    ROWS = CH * KW
    per_w = B // NW
    n_chunks = per_w // ROWS
    assert per_w % ROWS == 0 and n_chunks % 2 == 0
    H = n_chunks // 2
    idx_rows_w = per_w // KW

    mesh = plsc.VectorSubcoreMesh(core_axis_name="c", subcore_axis_name="s")

    @functools.partial(
        pl.kernel,
        mesh=mesh,
        out_type=jax.ShapeDtypeStruct((B, 2 * D), jnp.float32),
        scratch_types=[
            pltpu.VMEM((idx_rows_w, KW), jnp.int32),
            pltpu.VMEM((ROWS, D), jnp.float32),
            pltpu.VMEM((ROWS, D), jnp.float32),
            pltpu.SemaphoreType.DMA,
            pltpu.SemaphoreType.DMA,
            pltpu.SemaphoreType.DMA,
            pltpu.SemaphoreType.DMA,
        ],
        compiler_params=pltpu.CompilerParams(use_tc_tiling_on_sc=False),
    )
    def emb(idx_hbm, table_hbm, out_hbm, idx_v, rows0, rows1,
            g0, g1, o0, o1):
        wid = lax.axis_index("s") * NC + lax.axis_index("c")
        out_row0 = wid * per_w

        pltpu.sync_copy(idx_hbm.at[pl.ds(wid * idx_rows_w, idx_rows_w)],
                        idx_v)

        def fire_gathers(chunk, rows_v, sem):
            for i in range(CH):
                pltpu.make_async_copy(
                    table_hbm.at[idx_v.at[chunk * CH + i]],
                    rows_v.at[pl.ds(i * KW, KW), :],
                    sem,
                ).start()

        def drain_gathers(rows_v, sem):
            pltpu.make_async_copy(
                table_hbm.at[pl.ds(0, ROWS), :], rows_v, sem).wait()

        def fire_write(chunk, rows_v, sem):
            pltpu.make_async_copy(
                rows_v,
                out_hbm.at[pl.ds(out_row0 + chunk * ROWS, ROWS),
                           pl.ds(0, D)],
                sem,
            ).start()

        def drain_write(rows_v, sem):
            pltpu.make_async_copy(
                rows_v,
                out_hbm.at[pl.ds(0, ROWS), pl.ds(0, D)], sem).wait()

        fire_gathers(0, rows0, g0)
        fire_gathers(1, rows1, g1)

        def body(j, carry):
            drain_gathers(rows0, g0)
            fire_write(2 * j - 2, rows0, o0)
            drain_write(rows0, o0)
            fire_gathers(2 * j, rows0, g0)
            drain_gathers(rows1, g1)
            fire_write(2 * j - 1, rows1, o1)
            drain_write(rows1, o1)
            fire_gathers(2 * j + 1, rows1, g1)
            return carry

        lax.fori_loop(1, H, body, 0)

        drain_gathers(rows0, g0)
        fire_write(2 * H - 2, rows0, o0)
        drain_gathers(rows1, g1)
        fire_write(2 * H - 1, rows1, o1)
        drain_write(rows0, o0)
        drain_write(rows1, o1)

    return emb(idx2, table)


def kernel(input_tensor, weight):
    B0, B1 = input_tensor.shape
    V, D = weight.shape
    B = B0 * B1
    KW = 128
    NSPLIT = 10
    T1 = B1 // NSPLIT

    BLK = 32768
    HB = BLK // 2
    v = input_tensor.T.astype(jnp.int32)
    blk, off = v // BLK, v % BLK
    idx2 = (2 * (blk * HB + off % HB) + off // HB).reshape(B // KW, KW)
    split_rows = B // (NSPLIT * KW)

    w_t = weight.T
    table = _transpose_table(w_t)
    table2 = table.reshape(2 * table.shape[0], D)

    gs = [_sc_gather(idx2[h * split_rows:(h + 1) * split_rows],
                     table2, B // NSPLIT, D) for h in range(NSPLIT)]
    o = None
    for h in range(NSPLIT):
        o = _transpose_out(gs[h], h, o, T1, B1, B0, D)

    return jnp.transpose(o, (2, 0, 1))
